# baseline (speedup 1.0000x reference)
import functools

import jax
import jax.numpy as jnp
import numpy as np
from jax import lax
from jax.experimental import pallas as pl
from jax.experimental.pallas import tpu as pltpu
from jax.experimental.pallas import tpu_sc as plsc

NC = 2
NS = 16
NW = NC * NS

CHUNK = 128


def _pos_encoding(seq: int, dim: int) -> np.ndarray:
    pos = np.arange(seq, dtype=np.float32).reshape(-1, 1)
    div = np.exp(np.arange(0, dim, 2, dtype=np.float32) * -(np.log(10000.0) / dim)).astype(np.float32)
    pe = np.zeros((seq, dim), dtype=np.float32)
    ang = pos * div
    pe[:, 0::2] = np.sin(ang)
    pe[:, 1::2] = np.cos(ang)
    return pe


def _make_embed(seq: int, vocab: int, dim: int):
    assert seq % NW == 0
    rows_per_w = seq // NW
    assert rows_per_w % CHUNK == 0
    n_chunks = rows_per_w // CHUNK

    mesh = plsc.VectorSubcoreMesh(core_axis_name="c", subcore_axis_name="s")

    @functools.partial(
        pl.kernel,
        out_type=jax.ShapeDtypeStruct((seq, dim), jnp.float32),
        mesh=mesh,
        scratch_types=[
            pltpu.VMEM((n_chunks, CHUNK), jnp.int32),
            pltpu.VMEM((rows_per_w, dim), jnp.float32),
            pltpu.SemaphoreType.DMA,
        ],
        compiler_params=pltpu.CompilerParams(use_tc_tiling_on_sc=False),
    )
    def embed_kernel(table_hbm, idx_hbm, pe_hbm, out_hbm, idx_v, acc_v, sem):
        wid = lax.axis_index("s") * NC + lax.axis_index("c")
        base = wid * rows_per_w
        pltpu.sync_copy(idx_hbm.at[wid], idx_v)
        pltpu.sync_copy(pe_hbm.at[pl.ds(base, rows_per_w)], acc_v)
        copies = []
        for j in range(n_chunks):
            copies.append(
                pltpu.async_copy(
                    table_hbm.at[idx_v.at[j]],
                    acc_v.at[pl.ds(j * CHUNK, CHUNK)],
                    sem,
                    add=True,
                )
            )
        for c in copies:
            c.wait()
        pltpu.sync_copy(acc_v, out_hbm.at[pl.ds(base, rows_per_w)])

    return embed_kernel


def kernel(indices, table):
    seq = indices.shape[0]
    vocab, dim = table.shape
    rows_per_w = seq // NW
    pe = jnp.asarray(_pos_encoding(seq, dim))
    idx = jnp.reshape(indices.astype(jnp.int32), (NW, rows_per_w // CHUNK, CHUNK))
    return _make_embed(seq, vocab, dim)(table, idx, pe)
